# Initial kernel scaffold; baseline (speedup 1.0000x reference)
import jax
import jax.numpy as jnp
from jax.experimental import pallas as pl


def kernel(x_user, x_item, edge_index_social, edge_index_interacts, edge_index_rev_interacts, up_W, up_b, ip_W, ip_b, c1s_Wl, c1s_bl, c1s_Wr, c1i_Wl, c1i_bl, c1i_Wr, c1r_Wl, c1r_bl, c1r_Wr, c2s_Wl, c2s_bl, c2s_Wr, c2i_Wl, c2i_bl, c2i_Wr, c2r_Wl, c2r_bl, c2r_Wr):
    raise NotImplementedError("write your pallas kernel here")


# speedup vs baseline: 5.1257x; 5.1257x over previous
import functools

import jax
import jax.numpy as jnp
from jax import lax
from jax.experimental import pallas as pl
from jax.experimental.pallas import tpu as pltpu
from jax.experimental.pallas import tpu_sc as plsc

NU = 50000
NI = 50000
E = 800000
DIN = 128
H = 64
DOUT = 32

EP = 819200
ER = EP // 128
NP = 50048
DUMP = 50000
NTILE = 16
NB = 8
ROWS_T = ER // NTILE
NCH = ROWS_T // NB
ROWS_C = ER // 2 // NTILE
NCHC = ROWS_C // NB
TS = NP // NTILE

BLK = 2000


def _sc_segsum3(w2):
  mesh = plsc.VectorSubcoreMesh(core_axis_name="c", subcore_axis_name="s")
  out1 = jax.ShapeDtypeStruct((2 * NP, w2), jnp.float32)
  nb = 4 if w2 > 16 else NB
  nch = ROWS_T // nb

  @functools.partial(
      pl.kernel,
      out_type=(out1, out1, out1),
      mesh=mesh,
      compiler_params=pltpu.CompilerParams(use_tc_tiling_on_sc=False),
      scratch_types=[
          pltpu.VMEM((nb, 128), jnp.int32),
          pltpu.VMEM((nb, 128), jnp.int32),
          pltpu.VMEM((nb, 128, w2), jnp.float32),
          pltpu.VMEM_SHARED((NP, w2), jnp.float32),
          pltpu.SemaphoreType.DMA,
      ],
  )
  def k(ya, srca, dsta, yb, srcb, dstb, yc, srcc, dstc, zeros_hbm,
        outa, outb, outc, srcv, dstv, rows, acc, sem):
    c = lax.axis_index("c")
    s = lax.axis_index("s")
    for y, src2, dst2, out in ((ya, srca, dsta, outa),
                               (yb, srcb, dstb, outb),
                               (yc, srcc, dstc, outc)):
      pltpu.sync_copy(zeros_hbm.at[pl.ds(0, TS)], acc.at[pl.ds(s * TS, TS)])
      plsc.subcore_barrier()
      src_base = c * ER + s * ROWS_T
      dst_base = s * ROWS_T

      def chunk(i, carry):
        pltpu.sync_copy(src2.at[pl.ds(src_base + i * nb, nb)], srcv)
        pltpu.sync_copy(dst2.at[pl.ds(dst_base + i * nb, nb)], dstv)
        cps = [pltpu.async_copy(y.at[srcv.at[j]], rows.at[j], sem)
               for j in range(nb)]
        for cp in cps:
          cp.wait()
        for j in range(nb):
          pltpu.sync_copy(rows.at[j], acc.at[dstv.at[j]], add=True)
        return carry

      lax.fori_loop(0, nch, chunk, 0)
      plsc.subcore_barrier()
      pltpu.sync_copy(acc.at[pl.ds(s * TS, TS)],
                      out.at[pl.ds(c * NP + s * TS, TS)])
    return

  return k


def _sc_counts():
  mesh = plsc.VectorSubcoreMesh(core_axis_name="c", subcore_axis_name="s")
  out1 = jax.ShapeDtypeStruct((2 * NP, 16), jnp.float32)

  @functools.partial(
      pl.kernel,
      out_type=(out1, out1, out1),
      mesh=mesh,
      compiler_params=pltpu.CompilerParams(use_tc_tiling_on_sc=False),
      scratch_types=[
          pltpu.VMEM((NB, 128), jnp.int32),
          pltpu.VMEM((128, 16), jnp.float32),
          pltpu.VMEM_SHARED((NP, 16), jnp.float32),
          pltpu.SemaphoreType.DMA,
      ],
  )
  def k(dsta, dstb, dstc, ones_hbm, zeros_hbm,
        outa, outb, outc, dstv, ones, acc, sem):
    c = lax.axis_index("c")
    s = lax.axis_index("s")
    pltpu.sync_copy(ones_hbm, ones)
    for dst2, out in ((dsta, outa), (dstb, outb), (dstc, outc)):
      pltpu.sync_copy(zeros_hbm.at[pl.ds(0, TS)], acc.at[pl.ds(s * TS, TS)])
      plsc.subcore_barrier()
      base = c * (ER // 2) + s * ROWS_C

      def chunk(i, carry):
        pltpu.sync_copy(dst2.at[pl.ds(base + i * NB, NB)], dstv)
        for j in range(NB):
          pltpu.sync_copy(ones, acc.at[dstv.at[j]], add=True)
        return carry

      lax.fori_loop(0, NCHC, chunk, 0)
      plsc.subcore_barrier()
      pltpu.sync_copy(acc.at[pl.ds(s * TS, TS)],
                      out.at[pl.ds(c * NP + s * TS, TS)])
    return

  return k


_HI = lax.Precision.HIGHEST


def _full(shape):
  return pl.BlockSpec(shape, lambda i: (0,) * len(shape))


def _tc_pre(n, ny):
  grid = n // BLK
  in_specs = [pl.BlockSpec((BLK, DIN), lambda i: (i, 0)),
              _full((DIN, H)), _full((1, H))]
  in_specs += [_full((H, H))] * ny
  in_specs += [_full((H, H)), _full((1, H))]
  out_shape = tuple([jax.ShapeDtypeStruct((2, n, H // 2), jnp.float32)] * ny
                    + [jax.ShapeDtypeStruct((n, H), jnp.float32)])
  out_specs = tuple([pl.BlockSpec((2, BLK, H // 2), lambda i: (0, i, 0))] * ny
                    + [pl.BlockSpec((BLK, H), lambda i: (i, 0))])

  def body(*refs):
    x, pwt, pb = refs[0], refs[1], refs[2]
    wls = refs[3:3 + ny]
    wrt, blc = refs[3 + ny], refs[4 + ny]
    youts = refs[5 + ny:5 + 2 * ny]
    duo = refs[5 + 2 * ny]
    h = jnp.dot(x[...], pwt[...], preferred_element_type=jnp.float32,
                precision=_HI) + pb[...]
    for wl, yo in zip(wls, youts):
      yv = jnp.dot(h, wl[...], preferred_element_type=jnp.float32,
                   precision=_HI)
      yo[0] = yv[:, :H // 2]
      yo[1] = yv[:, H // 2:]
    duo[...] = jnp.dot(h, wrt[...], preferred_element_type=jnp.float32,
                       precision=_HI) + blc[...]

  return pl.pallas_call(body, grid=(grid,), in_specs=in_specs,
                        out_specs=out_specs, out_shape=out_shape)


def _agg(seg_ref, cnt_ref, w):
  inv = 1.0 / jnp.maximum(cnt_ref[0, :, 0:1] + cnt_ref[1, :, 0:1], 1.0)
  return jnp.concatenate([seg_ref[0], seg_ref[1]], axis=1) * inv


def _tc_mid(n, nrel, ny, w_in):
  grid = n // BLK
  in_specs = []
  for _ in range(nrel):
    in_specs += [pl.BlockSpec((2, BLK, w_in // 2), lambda i: (0, i, 0)),
                 pl.BlockSpec((2, BLK, 16), lambda i: (0, i, 0))]
  in_specs += [pl.BlockSpec((BLK, w_in), lambda i: (i, 0))]
  in_specs += [_full((w_in, DOUT))] * ny
  in_specs += [_full((w_in, DOUT)), _full((1, DOUT))]
  out_shape = tuple([jax.ShapeDtypeStruct((2, n, DOUT // 2), jnp.float32)] * ny
                    + [jax.ShapeDtypeStruct((n, DOUT), jnp.float32)])
  out_specs = tuple(
      [pl.BlockSpec((2, BLK, DOUT // 2), lambda i: (0, i, 0))] * ny
      + [pl.BlockSpec((BLK, DOUT), lambda i: (i, 0))])
  scale = 1.0 / nrel

  def body(*refs):
    pre = None
    for r in range(nrel):
      a = _agg(refs[2 * r], refs[2 * r + 1], w_in)
      pre = a if pre is None else pre + a
    d = refs[2 * nrel]
    wls = refs[2 * nrel + 1:2 * nrel + 1 + ny]
    wrt, blc = refs[2 * nrel + 1 + ny], refs[2 * nrel + 2 + ny]
    youts = refs[2 * nrel + 3 + ny:2 * nrel + 3 + 2 * ny]
    duo = refs[2 * nrel + 3 + 2 * ny]
    h1 = jnp.maximum((pre + d[...]) * scale, 0.0)
    for wl, yo in zip(wls, youts):
      yv = jnp.dot(h1, wl[...], preferred_element_type=jnp.float32,
                   precision=_HI)
      yo[0] = yv[:, :DOUT // 2]
      yo[1] = yv[:, DOUT // 2:]
    duo[...] = jnp.dot(h1, wrt[...], preferred_element_type=jnp.float32,
                       precision=_HI) + blc[...]

  return pl.pallas_call(body, grid=(grid,), in_specs=in_specs,
                        out_specs=out_specs, out_shape=out_shape)


def _tc_post(n, nrel):
  grid = n // BLK
  in_specs = []
  for _ in range(nrel):
    in_specs += [pl.BlockSpec((2, BLK, DOUT // 2), lambda i: (0, i, 0)),
                 pl.BlockSpec((2, BLK, 16), lambda i: (0, i, 0))]
  in_specs += [pl.BlockSpec((BLK, DOUT), lambda i: (i, 0))]
  out_shape = jax.ShapeDtypeStruct((n, DOUT), jnp.float32)
  out_specs = pl.BlockSpec((BLK, DOUT), lambda i: (i, 0))
  scale = 1.0 / nrel

  def body(*refs):
    pre = None
    for r in range(nrel):
      a = _agg(refs[2 * r], refs[2 * r + 1], DOUT)
      pre = a if pre is None else pre + a
    d = refs[2 * nrel]
    refs[2 * nrel + 1][...] = jnp.maximum((pre + d[...]) * scale, 0.0)

  return pl.pallas_call(body, grid=(grid,), in_specs=in_specs,
                        out_specs=out_specs, out_shape=out_shape)


def _prep_edges(ei, n_src):
  src = ei[0].astype(jnp.int32)
  dst = ei[1].astype(jnp.int32)
  pad = EP - E
  src = jnp.concatenate([src, jnp.zeros((pad,), jnp.int32)])
  dst = jnp.concatenate([dst, jnp.full((pad,), DUMP, jnp.int32)])
  src2 = jnp.concatenate([src, src + n_src]).reshape(2 * ER, 128)
  dst2 = dst.reshape(ER, 128)
  return src2, dst2


def kernel(x_user, x_item, edge_index_social, edge_index_interacts,
           edge_index_rev_interacts, up_W, up_b, ip_W, ip_b,
           c1s_Wl, c1s_bl, c1s_Wr, c1i_Wl, c1i_bl, c1i_Wr,
           c1r_Wl, c1r_bl, c1r_Wr,
           c2s_Wl, c2s_bl, c2s_Wr, c2i_Wl, c2i_bl, c2i_Wr,
           c2r_Wl, c2r_bl, c2r_Wr):
  src2_s, dst2_s = _prep_edges(edge_index_social, NU)
  src2_i, dst2_i = _prep_edges(edge_index_interacts, NU)
  src2_r, dst2_r = _prep_edges(edge_index_rev_interacts, NI)

  zeros32 = jnp.zeros((TS, H // 2), jnp.float32)
  zeros16 = jnp.zeros((TS, 16), jnp.float32)
  ones128 = jnp.ones((128, 16), jnp.float32)

  ys1, yi1, du1 = _tc_pre(NU, 2)(
      x_user, up_W.T, up_b.reshape(1, H),
      c1s_Wl.T, c1i_Wl.T,
      (c1s_Wr + c1r_Wr).T, (c1s_bl + c1r_bl).reshape(1, H))
  yr1, di1 = _tc_pre(NI, 1)(
      x_item, ip_W.T, ip_b.reshape(1, H),
      c1r_Wl.T,
      c1i_Wr.T, c1i_bl.reshape(1, H))

  cnt_s, cnt_i, cnt_r = _sc_counts()(
      dst2_s, dst2_i, dst2_r, ones128, zeros16)
  seg_s, seg_i, seg_r = _sc_segsum3(H // 2)(
      ys1.reshape(2 * NU, H // 2), src2_s, dst2_s,
      yi1.reshape(2 * NU, H // 2), src2_i, dst2_i,
      yr1.reshape(2 * NI, H // 2), src2_r, dst2_r,
      zeros32)

  ys2, yi2, du2 = _tc_mid(NU, 2, 2, H)(
      seg_s.reshape(2, NP, H // 2), cnt_s.reshape(2, NP, 16),
      seg_r.reshape(2, NP, H // 2), cnt_r.reshape(2, NP, 16),
      du1,
      c2s_Wl.T, c2i_Wl.T,
      (c2s_Wr + c2r_Wr).T, (c2s_bl + c2r_bl).reshape(1, DOUT))
  yr2, di2 = _tc_mid(NI, 1, 1, H)(
      seg_i.reshape(2, NP, H // 2), cnt_i.reshape(2, NP, 16),
      di1,
      c2r_Wl.T,
      c2i_Wr.T, c2i_bl.reshape(1, DOUT))

  seg_s2, seg_i2, seg_r2 = _sc_segsum3(DOUT // 2)(
      ys2.reshape(2 * NU, DOUT // 2), src2_s, dst2_s,
      yi2.reshape(2 * NU, DOUT // 2), src2_i, dst2_i,
      yr2.reshape(2 * NI, DOUT // 2), src2_r, dst2_r,
      zeros16)

  h2u = _tc_post(NU, 2)(
      seg_s2.reshape(2, NP, DOUT // 2), cnt_s.reshape(2, NP, 16),
      seg_r2.reshape(2, NP, DOUT // 2), cnt_r.reshape(2, NP, 16),
      du2)
  h2i = _tc_post(NI, 1)(
      seg_i2.reshape(2, NP, DOUT // 2), cnt_i.reshape(2, NP, 16),
      di2)
  return (h2u, h2i)


# speedup vs baseline: 5.7267x; 1.1172x over previous
import functools

import jax
import jax.numpy as jnp
from jax import lax
from jax.experimental import pallas as pl
from jax.experimental.pallas import tpu as pltpu
from jax.experimental.pallas import tpu_sc as plsc

NU = 50000
NI = 50000
E = 800000
DIN = 128
H = 64
DOUT = 32

EP = 819200
ER = EP // 128
NP = 50048
DUMP = 50000
NTILE = 16
NB = 8
ROWS_T = ER // NTILE
NCH = ROWS_T // NB
ROWS_C = ER // 2 // NTILE
NBC = 10
NCHC = ROWS_C // NBC
TS = NP // NTILE

BLK = 2000


def _sc_segsum3(w2):
  mesh = plsc.VectorSubcoreMesh(core_axis_name="c", subcore_axis_name="s")
  out1 = jax.ShapeDtypeStruct((2 * NP, w2), jnp.float32)
  nb = 2 if w2 > 16 else NB
  nch = ROWS_T // nb
  buf = lambda: [pltpu.VMEM((nb, 128), jnp.int32),
                 pltpu.VMEM((nb, 128), jnp.int32),
                 pltpu.VMEM((nb, 128, w2), jnp.float32),
                 pltpu.SemaphoreType.DMA,
                 pltpu.SemaphoreType.DMA]

  @functools.partial(
      pl.kernel,
      out_type=(out1, out1, out1),
      mesh=mesh,
      compiler_params=pltpu.CompilerParams(use_tc_tiling_on_sc=False),
      scratch_types=buf() + buf() + [
          pltpu.VMEM_SHARED((NP, w2), jnp.float32),
      ],
  )
  def k(ya, srca, dsta, yb, srcb, dstb, yc, srcc, dstc, zeros_hbm,
        outa, outb, outc,
        srcv0, dstv0, rows0, gsem0, ssem0,
        srcv1, dstv1, rows1, gsem1, ssem1, acc):
    c = lax.axis_index("c")
    s = lax.axis_index("s")
    bufs = ((srcv0, dstv0, rows0, gsem0, ssem0),
            (srcv1, dstv1, rows1, gsem1, ssem1))

    def drain_sc(rows, ssem):
      for j in range(nb):
        pltpu.make_async_copy(rows.at[j], acc.at[pl.ds(0, 128)], ssem).wait()

    for y, src2, dst2, out in ((ya, srca, dsta, outa),
                               (yb, srcb, dstb, outb),
                               (yc, srcc, dstc, outc)):
      pltpu.sync_copy(zeros_hbm.at[pl.ds(0, TS)], acc.at[pl.ds(s * TS, TS)])
      plsc.subcore_barrier()
      src_base = c * ER + s * ROWS_T
      dst_base = s * ROWS_T

      def pair(i2, carry):
        gcps = []
        for b, (srcv, dstv, rows, gsem, ssem) in enumerate(bufs):
          ci = 2 * i2 + b

          @pl.when(i2 > 0)
          def _():
            drain_sc(rows, ssem)

          pltpu.sync_copy(src2.at[pl.ds(src_base + ci * nb, nb)], srcv)
          pltpu.sync_copy(dst2.at[pl.ds(dst_base + ci * nb, nb)], dstv)
          gcps.append([pltpu.async_copy(y.at[srcv.at[j]], rows.at[j], gsem)
                       for j in range(nb)])
        for b, (srcv, dstv, rows, gsem, ssem) in enumerate(bufs):
          for cp in gcps[b]:
            cp.wait()
          for j in range(nb):
            pltpu.async_copy(rows.at[j], acc.at[dstv.at[j]], ssem, add=True)
        return carry

      lax.fori_loop(0, nch // 2, pair, 0)
      for (srcv, dstv, rows, gsem, ssem) in bufs:
        drain_sc(rows, ssem)
      plsc.subcore_barrier()
      pltpu.sync_copy(acc.at[pl.ds(s * TS, TS)],
                      out.at[pl.ds(c * NP + s * TS, TS)])
    return

  return k


def _sc_counts():
  mesh = plsc.VectorSubcoreMesh(core_axis_name="c", subcore_axis_name="s")
  out1 = jax.ShapeDtypeStruct((2 * NP, 16), jnp.float32)

  @functools.partial(
      pl.kernel,
      out_type=(out1, out1, out1),
      mesh=mesh,
      compiler_params=pltpu.CompilerParams(use_tc_tiling_on_sc=False),
      scratch_types=[
          pltpu.VMEM((NBC, 128), jnp.int32),
          pltpu.VMEM((NBC, 128), jnp.int32),
          pltpu.VMEM((128, 16), jnp.float32),
          pltpu.VMEM_SHARED((NP, 16), jnp.float32),
          pltpu.SemaphoreType.DMA,
          pltpu.SemaphoreType.DMA,
      ],
  )
  def k(dsta, dstb, dstc, ones_hbm, zeros_hbm,
        outa, outb, outc, dstv0, dstv1, ones, acc, sem0, sem1):
    c = lax.axis_index("c")
    s = lax.axis_index("s")
    pltpu.sync_copy(ones_hbm, ones)
    bufs = ((dstv0, sem0), (dstv1, sem1))

    def drain(sem):
      for j in range(NBC):
        pltpu.make_async_copy(ones, acc.at[pl.ds(0, 128)], sem).wait()

    for dst2, out in ((dsta, outa), (dstb, outb), (dstc, outc)):
      pltpu.sync_copy(zeros_hbm.at[pl.ds(0, TS)], acc.at[pl.ds(s * TS, TS)])
      plsc.subcore_barrier()
      base = c * (ER // 2) + s * ROWS_C

      def pair(i2, carry):
        for b, (dstv, sem) in enumerate(bufs):
          @pl.when(i2 > 0)
          def _():
            drain(sem)

          pltpu.sync_copy(dst2.at[pl.ds(base + (2 * i2 + b) * NBC, NBC)], dstv)
          for j in range(NBC):
            pltpu.async_copy(ones, acc.at[dstv.at[j]], sem, add=True)
        return carry

      lax.fori_loop(0, NCHC // 2, pair, 0)
      for dstv, sem in bufs:
        drain(sem)
      plsc.subcore_barrier()
      pltpu.sync_copy(acc.at[pl.ds(s * TS, TS)],
                      out.at[pl.ds(c * NP + s * TS, TS)])
    return

  return k


_HI = lax.Precision.HIGHEST


def _full(shape):
  return pl.BlockSpec(shape, lambda i: (0,) * len(shape))


def _tc_pre(n, ny):
  grid = n // BLK
  in_specs = [pl.BlockSpec((BLK, DIN), lambda i: (i, 0)),
              _full((DIN, H)), _full((1, H))]
  in_specs += [_full((H, H))] * ny
  in_specs += [_full((H, H)), _full((1, H))]
  out_shape = tuple([jax.ShapeDtypeStruct((2, n, H // 2), jnp.float32)] * ny
                    + [jax.ShapeDtypeStruct((n, H), jnp.float32)])
  out_specs = tuple([pl.BlockSpec((2, BLK, H // 2), lambda i: (0, i, 0))] * ny
                    + [pl.BlockSpec((BLK, H), lambda i: (i, 0))])

  def body(*refs):
    x, pwt, pb = refs[0], refs[1], refs[2]
    wls = refs[3:3 + ny]
    wrt, blc = refs[3 + ny], refs[4 + ny]
    youts = refs[5 + ny:5 + 2 * ny]
    duo = refs[5 + 2 * ny]
    h = jnp.dot(x[...], pwt[...], preferred_element_type=jnp.float32,
                precision=_HI) + pb[...]
    for wl, yo in zip(wls, youts):
      yv = jnp.dot(h, wl[...], preferred_element_type=jnp.float32,
                   precision=_HI)
      yo[0] = yv[:, :H // 2]
      yo[1] = yv[:, H // 2:]
    duo[...] = jnp.dot(h, wrt[...], preferred_element_type=jnp.float32,
                       precision=_HI) + blc[...]

  return pl.pallas_call(body, grid=(grid,), in_specs=in_specs,
                        out_specs=out_specs, out_shape=out_shape)


def _agg(seg_ref, cnt_ref, w):
  inv = 1.0 / jnp.maximum(cnt_ref[0, :, 0:1] + cnt_ref[1, :, 0:1], 1.0)
  return jnp.concatenate([seg_ref[0], seg_ref[1]], axis=1) * inv


def _tc_mid(n, nrel, ny, w_in):
  grid = n // BLK
  in_specs = []
  for _ in range(nrel):
    in_specs += [pl.BlockSpec((2, BLK, w_in // 2), lambda i: (0, i, 0)),
                 pl.BlockSpec((2, BLK, 16), lambda i: (0, i, 0))]
  in_specs += [pl.BlockSpec((BLK, w_in), lambda i: (i, 0))]
  in_specs += [_full((w_in, DOUT))] * ny
  in_specs += [_full((w_in, DOUT)), _full((1, DOUT))]
  out_shape = tuple([jax.ShapeDtypeStruct((2, n, DOUT // 2), jnp.float32)] * ny
                    + [jax.ShapeDtypeStruct((n, DOUT), jnp.float32)])
  out_specs = tuple(
      [pl.BlockSpec((2, BLK, DOUT // 2), lambda i: (0, i, 0))] * ny
      + [pl.BlockSpec((BLK, DOUT), lambda i: (i, 0))])
  scale = 1.0 / nrel

  def body(*refs):
    pre = None
    for r in range(nrel):
      a = _agg(refs[2 * r], refs[2 * r + 1], w_in)
      pre = a if pre is None else pre + a
    d = refs[2 * nrel]
    wls = refs[2 * nrel + 1:2 * nrel + 1 + ny]
    wrt, blc = refs[2 * nrel + 1 + ny], refs[2 * nrel + 2 + ny]
    youts = refs[2 * nrel + 3 + ny:2 * nrel + 3 + 2 * ny]
    duo = refs[2 * nrel + 3 + 2 * ny]
    h1 = jnp.maximum((pre + d[...]) * scale, 0.0)
    for wl, yo in zip(wls, youts):
      yv = jnp.dot(h1, wl[...], preferred_element_type=jnp.float32,
                   precision=_HI)
      yo[0] = yv[:, :DOUT // 2]
      yo[1] = yv[:, DOUT // 2:]
    duo[...] = jnp.dot(h1, wrt[...], preferred_element_type=jnp.float32,
                       precision=_HI) + blc[...]

  return pl.pallas_call(body, grid=(grid,), in_specs=in_specs,
                        out_specs=out_specs, out_shape=out_shape)


def _tc_post(n, nrel):
  grid = n // BLK
  in_specs = []
  for _ in range(nrel):
    in_specs += [pl.BlockSpec((2, BLK, DOUT // 2), lambda i: (0, i, 0)),
                 pl.BlockSpec((2, BLK, 16), lambda i: (0, i, 0))]
  in_specs += [pl.BlockSpec((BLK, DOUT), lambda i: (i, 0))]
  out_shape = jax.ShapeDtypeStruct((n, DOUT), jnp.float32)
  out_specs = pl.BlockSpec((BLK, DOUT), lambda i: (i, 0))
  scale = 1.0 / nrel

  def body(*refs):
    pre = None
    for r in range(nrel):
      a = _agg(refs[2 * r], refs[2 * r + 1], DOUT)
      pre = a if pre is None else pre + a
    d = refs[2 * nrel]
    refs[2 * nrel + 1][...] = jnp.maximum((pre + d[...]) * scale, 0.0)

  return pl.pallas_call(body, grid=(grid,), in_specs=in_specs,
                        out_specs=out_specs, out_shape=out_shape)


def _prep_edges(ei, n_src):
  src = ei[0].astype(jnp.int32)
  dst = ei[1].astype(jnp.int32)
  pad = EP - E
  src = jnp.concatenate([src, jnp.zeros((pad,), jnp.int32)])
  dst = jnp.concatenate([dst, jnp.full((pad,), DUMP, jnp.int32)])
  src2 = jnp.concatenate([src, src + n_src]).reshape(2 * ER, 128)
  dst2 = dst.reshape(ER, 128)
  return src2, dst2


def kernel(x_user, x_item, edge_index_social, edge_index_interacts,
           edge_index_rev_interacts, up_W, up_b, ip_W, ip_b,
           c1s_Wl, c1s_bl, c1s_Wr, c1i_Wl, c1i_bl, c1i_Wr,
           c1r_Wl, c1r_bl, c1r_Wr,
           c2s_Wl, c2s_bl, c2s_Wr, c2i_Wl, c2i_bl, c2i_Wr,
           c2r_Wl, c2r_bl, c2r_Wr):
  src2_s, dst2_s = _prep_edges(edge_index_social, NU)
  src2_i, dst2_i = _prep_edges(edge_index_interacts, NU)
  src2_r, dst2_r = _prep_edges(edge_index_rev_interacts, NI)

  zeros32 = jnp.zeros((TS, H // 2), jnp.float32)
  zeros16 = jnp.zeros((TS, 16), jnp.float32)
  ones128 = jnp.ones((128, 16), jnp.float32)

  ys1, yi1, du1 = _tc_pre(NU, 2)(
      x_user, up_W.T, up_b.reshape(1, H),
      c1s_Wl.T, c1i_Wl.T,
      (c1s_Wr + c1r_Wr).T, (c1s_bl + c1r_bl).reshape(1, H))
  yr1, di1 = _tc_pre(NI, 1)(
      x_item, ip_W.T, ip_b.reshape(1, H),
      c1r_Wl.T,
      c1i_Wr.T, c1i_bl.reshape(1, H))

  cnt_s, cnt_i, cnt_r = _sc_counts()(
      dst2_s, dst2_i, dst2_r, ones128, zeros16)
  seg_s, seg_i, seg_r = _sc_segsum3(H // 2)(
      ys1.reshape(2 * NU, H // 2), src2_s, dst2_s,
      yi1.reshape(2 * NU, H // 2), src2_i, dst2_i,
      yr1.reshape(2 * NI, H // 2), src2_r, dst2_r,
      zeros32)

  ys2, yi2, du2 = _tc_mid(NU, 2, 2, H)(
      seg_s.reshape(2, NP, H // 2), cnt_s.reshape(2, NP, 16),
      seg_r.reshape(2, NP, H // 2), cnt_r.reshape(2, NP, 16),
      du1,
      c2s_Wl.T, c2i_Wl.T,
      (c2s_Wr + c2r_Wr).T, (c2s_bl + c2r_bl).reshape(1, DOUT))
  yr2, di2 = _tc_mid(NI, 1, 1, H)(
      seg_i.reshape(2, NP, H // 2), cnt_i.reshape(2, NP, 16),
      di1,
      c2r_Wl.T,
      c2i_Wr.T, c2i_bl.reshape(1, DOUT))

  seg_s2, seg_i2, seg_r2 = _sc_segsum3(DOUT // 2)(
      ys2.reshape(2 * NU, DOUT // 2), src2_s, dst2_s,
      yi2.reshape(2 * NU, DOUT // 2), src2_i, dst2_i,
      yr2.reshape(2 * NI, DOUT // 2), src2_r, dst2_r,
      zeros16)

  h2u = _tc_post(NU, 2)(
      seg_s2.reshape(2, NP, DOUT // 2), cnt_s.reshape(2, NP, 16),
      seg_r2.reshape(2, NP, DOUT // 2), cnt_r.reshape(2, NP, 16),
      du2)
  h2i = _tc_post(NI, 1)(
      seg_i2.reshape(2, NP, DOUT // 2), cnt_i.reshape(2, NP, 16),
      di2)
  return (h2u, h2i)
